# Initial kernel scaffold; baseline (speedup 1.0000x reference)
import jax
import jax.numpy as jnp
from jax.experimental import pallas as pl


def kernel(lngs, lats, grid_id, emb_table, W, b):
    raise NotImplementedError("write your pallas kernel here")


# speedup vs baseline: 2.5596x; 2.5596x over previous
import functools

import jax
import jax.numpy as jnp
from jax import lax
from jax.experimental import pallas as pl
from jax.experimental.pallas import tpu as pltpu
from jax.experimental.pallas import tpu_sc as plsc

_B, _L = 4096, 200
_V, _D = 128 * 128, 32
_N = _B * _L

_NC, _NS, _LANES = 2, 16, 16
_NW = _NC * _NS
_RPW = _N // _NW
_CH = 1024
_NCHUNK = _RPW // _CH
_G = _CH // 128


def _table_transform(emb_table, w2, b2):
    def body(emb_ref, w_ref, b_ref, out_ref):
        out_ref[...] = (
            jnp.dot(emb_ref[...], w_ref[...], preferred_element_type=jnp.float32,
                    precision=jax.lax.Precision.HIGHEST)
            + b_ref[...]
        )

    return pl.pallas_call(
        body,
        out_shape=jax.ShapeDtypeStruct((_V, _D), jnp.float32),
    )(emb_table, w2, b2)


def _tanh16(y):
    t = jnp.exp(y * 2.0)
    return 1.0 - 2.0 / (t + 1.0)


def _sc_lookup(table, idx2d, lng_f, lat_f, w01):
    mesh = plsc.VectorSubcoreMesh(core_axis_name="c", subcore_axis_name="s")

    @functools.partial(
        pl.kernel,
        mesh=mesh,
        out_type=jax.ShapeDtypeStruct((_N, _D), jnp.float32),
        scratch_types=[
            pltpu.VMEM((_G, 128), jnp.int32),
            pltpu.VMEM((_CH,), jnp.float32),
            pltpu.VMEM((_CH,), jnp.float32),
            pltpu.VMEM((_CH, _D), jnp.float32),
            pltpu.VMEM((2, _D), jnp.float32),
            pltpu.SemaphoreType.DMA,
        ],
        compiler_params=pltpu.CompilerParams(
            needs_layout_passes=False, use_tc_tiling_on_sc=False
        ),
    )
    def k(table_hbm, idx_hbm, lng_hbm, lat_hbm, w01_hbm, out_hbm,
          idx_v, lng_v, lat_v, rows_v, w01_v, sem):
        wid = lax.axis_index("s") * _NC + lax.axis_index("c")
        base = wid * _RPW

        pltpu.sync_copy(w01_hbm, w01_v)
        w0a = w01_v[0, pl.ds(0, _LANES)]
        w0b = w01_v[0, pl.ds(_LANES, _LANES)]
        w1a = w01_v[1, pl.ds(0, _LANES)]
        w1b = w01_v[1, pl.ds(_LANES, _LANES)]

        def chunk_body(c, carry):
            r0 = pl.multiple_of(base + c * _CH, _CH)
            i0 = pl.multiple_of(r0 // 128, _G)
            pltpu.sync_copy(idx_hbm.at[pl.ds(i0, _G), :], idx_v)
            pltpu.sync_copy(lng_hbm.at[pl.ds(r0, _CH)], lng_v)
            pltpu.sync_copy(lat_hbm.at[pl.ds(r0, _CH)], lat_v)
            cps = [
                pltpu.async_copy(
                    table_hbm.at[idx_v.at[j]],
                    rows_v.at[pl.ds(j * 128, 128), :],
                    sem,
                )
                for j in range(_G)
            ]
            for cp in cps:
                cp.wait()

            def row_body(r, rcarry):
                rv = jnp.full((_LANES,), r, jnp.int32)
                lng = plsc.load_gather(lng_v, [rv])
                lat = plsc.load_gather(lat_v, [rv])
                g0 = rows_v[r, pl.ds(0, _LANES)]
                g1 = rows_v[r, pl.ds(_LANES, _LANES)]
                y0 = g0 + lng * w0a + lat * w1a
                y1 = g1 + lng * w0b + lat * w1b
                rows_v[r, pl.ds(0, _LANES)] = _tanh16(y0)
                rows_v[r, pl.ds(_LANES, _LANES)] = _tanh16(y1)
                return rcarry

            lax.fori_loop(0, _CH, row_body, 0, unroll=2)
            pltpu.sync_copy(rows_v, out_hbm.at[pl.ds(r0, _CH), :])
            return carry

        lax.fori_loop(0, _NCHUNK, chunk_body, 0)

    return k(table, idx2d, lng_f, lat_f, w01)


def kernel(lngs, lats, grid_id, emb_table, W, b):
    table = _table_transform(emb_table, W[2:], b.reshape(1, _D))
    idx2d = grid_id.reshape(_N // 128, 128).astype(jnp.int32)
    lng_f = lngs.reshape(_N)
    lat_f = lats.reshape(_N)
    out = _sc_lookup(table, idx2d, lng_f, lat_f, W[:2])
    return out.reshape(_B, _L, _D)
